# Initial kernel scaffold; baseline (speedup 1.0000x reference)
import jax
import jax.numpy as jnp
from jax.experimental import pallas as pl


def kernel(xyz, features, W0, g0, b0, W1, g1, b1):
    raise NotImplementedError("write your pallas kernel here")


# speedup vs baseline: 14.6587x; 14.6587x over previous
import functools

import jax
import jax.numpy as jnp
from jax import lax
from jax.experimental import pallas as pl
from jax.experimental.pallas import tpu as pltpu
from jax.experimental.pallas import tpu_sc as plsc

B = 4
N = 8192
S = 512
NS = 32
R2 = 0.2 * 0.2
CPAD = 32

NC = 2
NSUB = 16
NW = NC * NSUB
ROWS = B * S * NS
RPW = ROWS // NW
GCH = 128
NCHUNK = RPW // GCH


def _fps_body(xyz_ref, inds_ref, nx_ref):
    X = xyz_ref[0]
    Y = xyz_ref[1]
    Z = xyz_ref[2]
    niota = lax.broadcasted_iota(jnp.int32, (B, N), 1)
    siota = lax.broadcasted_iota(jnp.int32, (B, S), 1)

    def body(i, carry):
        dists, far, inds, nxx, nxy, nxz = carry
        m = niota == far
        cx = jnp.sum(jnp.where(m, X, 0.0), axis=1, keepdims=True)
        cy = jnp.sum(jnp.where(m, Y, 0.0), axis=1, keepdims=True)
        cz = jnp.sum(jnp.where(m, Z, 0.0), axis=1, keepdims=True)
        smi = (siota == i).astype(jnp.int32)
        smf = smi.astype(jnp.float32)
        inds = inds + smi * far
        nxx = nxx + smf * cx
        nxy = nxy + smf * cy
        nxz = nxz + smf * cz
        d = (X - cx) ** 2 + (Y - cy) ** 2 + (Z - cz) ** 2
        dists = jnp.minimum(dists, d)
        mx = jnp.max(dists, axis=1, keepdims=True)
        c = (dists == mx).astype(jnp.int32)
        far = jnp.min(N + c * (niota - N), axis=1, keepdims=True)
        return dists, far, inds, nxx, nxy, nxz

    riota = lax.broadcasted_iota(jnp.int32, (B, S), 0)
    zi = (riota + siota) >> 31
    zf = zi.astype(jnp.float32)
    rn = lax.broadcasted_iota(jnp.int32, (B, N), 0)
    init = (
        ((rn + niota) >> 31).astype(jnp.float32) + 1e10,
        (rn[:, :1] + niota[:, :1]) >> 31,
        zi,
        zf,
        zf,
        zf,
    )
    _, _, inds, nxx, nxy, nxz = lax.fori_loop(0, S, body, init)
    inds_ref[...] = inds
    nx_ref[0] = nxx
    nx_ref[1] = nxy
    nx_ref[2] = nxz


def _run_fps(xyz_t):
    return pl.pallas_call(
        _fps_body,
        out_shape=[
            jax.ShapeDtypeStruct((B, S), jnp.int32),
            jax.ShapeDtypeStruct((3, B, S), jnp.float32),
        ],
    )(xyz_t)


TS = 128


def _bq_body(xyz_ref, nx_ref, out_ref):
    b = pl.program_id(0)
    X = xyz_ref[0, 0:1, :]
    Y = xyz_ref[0, 1:2, :]
    Z = xyz_ref[0, 2:3, :]
    cx = nx_ref[0, :, 0:1]
    cy = nx_ref[0, :, 1:2]
    cz = nx_ref[0, :, 2:3]
    c2 = cx * cx + cy * cy + cz * cz
    x2 = X * X + Y * Y + Z * Z
    e = jnp.dot(nx_ref[0], xyz_ref[0],
                preferred_element_type=jnp.float32)
    d = (c2 + x2) - 2.0 * e
    niota = lax.broadcasted_iota(jnp.int32, (TS, N), 1)
    inr = (d <= R2).astype(jnp.int32)
    key = N + inr * (niota - N)
    goff = b * N
    cur = jnp.min(key, axis=1, keepdims=True)
    f0 = jnp.where(cur == N, 0, cur)
    out_ref[0, :, 0:1] = f0 + goff
    prev = cur
    for k in range(1, NS):
        cand = jnp.min(jnp.where(key > prev, key, N), axis=1, keepdims=True)
        out_ref[0, :, k:k + 1] = jnp.where(cand == N, f0, cand) + goff
        prev = cand


def _run_ball_query(xyz_bt, nx3):
    return pl.pallas_call(
        _bq_body,
        grid=(B, S // TS),
        in_specs=[
            pl.BlockSpec((1, 3, N), lambda b, st: (b, 0, 0)),
            pl.BlockSpec((1, TS, 3), lambda b, st: (b, st, 0)),
        ],
        out_specs=pl.BlockSpec((1, TS, NS), lambda b, st: (b, st, 0)),
        out_shape=jax.ShapeDtypeStruct((B, S, NS), jnp.int32),
    )(xyz_bt, nx3)


def _sc_gather_body(tf_hbm, idx_hbm, out_hbm, idx_v, rows_v, sem):
    wid = lax.axis_index("s") * NC + lax.axis_index("c")
    pltpu.sync_copy(idx_hbm.at[wid], idx_v)
    descs = []
    for c in range(NCHUNK):
        descs.append(
            pltpu.async_copy(
                tf_hbm.at[idx_v.at[c]],
                rows_v.at[pl.ds(c * GCH, GCH)],
                sem,
            )
        )
    for d in descs:
        d.wait()
    pltpu.sync_copy(rows_v, out_hbm.at[pl.ds(wid * RPW, RPW)])


def _run_sc_gather(tf, idx3):
    mesh = plsc.VectorSubcoreMesh(core_axis_name="c", subcore_axis_name="s")
    k = pl.kernel(
        _sc_gather_body,
        out_type=jax.ShapeDtypeStruct((ROWS, CPAD), jnp.float32),
        mesh=mesh,
        scratch_types=[
            pltpu.VMEM((NCHUNK, GCH), jnp.int32),
            pltpu.VMEM((RPW, CPAD), jnp.float32),
            pltpu.SemaphoreType.DMA,
        ],
        compiler_params=pltpu.CompilerParams(use_tc_tiling_on_sc=False),
    )
    return k(tf, idx3)


CH = 8192
NCH = ROWS // CH


def _d1_body(g_ref, nx_ref, w_ref, z_ref, s_ref, q_ref):
    i = pl.program_id(0)
    dg = g_ref[...] - nx_ref[...]
    z = jnp.dot(dg, w_ref[...], preferred_element_type=jnp.float32)
    z_ref[...] = z

    @pl.when(i == 0)
    def _():
        s_ref[...] = jnp.zeros_like(s_ref)
        q_ref[...] = jnp.zeros_like(q_ref)

    s_ref[...] += jnp.sum(z, axis=0, keepdims=True)
    q_ref[...] += jnp.sum(z * z, axis=0, keepdims=True)


def _run_d1(g, nxrep, w0et):
    return pl.pallas_call(
        _d1_body,
        grid=(NCH,),
        in_specs=[
            pl.BlockSpec((CH, CPAD), lambda i: (i, 0)),
            pl.BlockSpec((CH, CPAD), lambda i: (i, 0)),
            pl.BlockSpec((CPAD, 32), lambda i: (0, 0)),
        ],
        out_specs=[
            pl.BlockSpec((CH, 32), lambda i: (i, 0)),
            pl.BlockSpec((1, 32), lambda i: (0, 0)),
            pl.BlockSpec((1, 32), lambda i: (0, 0)),
        ],
        out_shape=[
            jax.ShapeDtypeStruct((ROWS, 32), jnp.float32),
            jax.ShapeDtypeStruct((1, 32), jnp.float32),
            jax.ShapeDtypeStruct((1, 32), jnp.float32),
        ],
    )(g, nxrep, w0et)


def _d2_body(z_ref, a_ref, c_ref, w_ref, z1_ref, s_ref, q_ref):
    i = pl.program_id(0)
    x = jnp.maximum(z_ref[...] * a_ref[...] + c_ref[...], 0.0)
    z1 = jnp.dot(x, w_ref[...], preferred_element_type=jnp.float32)
    z1_ref[...] = z1

    @pl.when(i == 0)
    def _():
        s_ref[...] = jnp.zeros_like(s_ref)
        q_ref[...] = jnp.zeros_like(q_ref)

    s_ref[...] += jnp.sum(z1, axis=0, keepdims=True)
    q_ref[...] += jnp.sum(z1 * z1, axis=0, keepdims=True)


def _run_d2(z0, a0, c0, w1t):
    return pl.pallas_call(
        _d2_body,
        grid=(NCH,),
        in_specs=[
            pl.BlockSpec((CH, 32), lambda i: (i, 0)),
            pl.BlockSpec((1, 32), lambda i: (0, 0)),
            pl.BlockSpec((1, 32), lambda i: (0, 0)),
            pl.BlockSpec((32, 64), lambda i: (0, 0)),
        ],
        out_specs=[
            pl.BlockSpec((CH, 64), lambda i: (i, 0)),
            pl.BlockSpec((1, 64), lambda i: (0, 0)),
            pl.BlockSpec((1, 64), lambda i: (0, 0)),
        ],
        out_shape=[
            jax.ShapeDtypeStruct((ROWS, 64), jnp.float32),
            jax.ShapeDtypeStruct((1, 64), jnp.float32),
            jax.ShapeDtypeStruct((1, 64), jnp.float32),
        ],
    )(z0, a0, c0, w1t)


def _d3_body(z_ref, a_ref, c_ref, o_ref):
    x = jnp.maximum(z_ref[...] * a_ref[...] + c_ref[...], 0.0)
    o_ref[...] = jnp.max(x, axis=1)


SCH = CH // NS


def _run_d3(z1g, a1, c1):
    return pl.pallas_call(
        _d3_body,
        grid=(NCH,),
        in_specs=[
            pl.BlockSpec((SCH, NS, 64), lambda i: (i, 0, 0)),
            pl.BlockSpec((1, 1, 64), lambda i: (0, 0, 0)),
            pl.BlockSpec((1, 1, 64), lambda i: (0, 0, 0)),
        ],
        out_specs=pl.BlockSpec((SCH, 64), lambda i: (i, 0)),
        out_shape=jax.ShapeDtypeStruct((B * S, 64), jnp.float32),
    )(z1g, a1, c1)


def _bn_coeffs(s, q, g, b, n):
    mean = s / n
    var = q / n - mean * mean
    a = g / jnp.sqrt(var + 1e-5)
    c = b - mean * a
    return a, c


@jax.jit
def kernel(xyz, features, W0, g0, b0, W1, g1, b1):
    xyz_t = jnp.transpose(xyz, (2, 0, 1))
    inds, nx = _run_fps(xyz_t)
    nx3 = jnp.transpose(nx, (1, 2, 0))

    idx = _run_ball_query(jnp.transpose(xyz, (0, 2, 1)), nx3)
    idx3 = idx.reshape(NW, NCHUNK, GCH)

    table = jnp.concatenate(
        [xyz, features, jnp.zeros((B, N, CPAD - 3 - features.shape[-1]), jnp.float32)],
        axis=-1,
    ).reshape(B * N, CPAD)
    g = _run_sc_gather(table, idx3)

    nxrep = jnp.concatenate(
        [nx3, jnp.zeros((B, S, CPAD - 3), jnp.float32)], axis=-1
    )
    nxrep = jnp.broadcast_to(nxrep[:, :, None, :], (B, S, NS, CPAD)).reshape(
        ROWS, CPAD
    )

    w0et = jnp.zeros((CPAD, 32), jnp.float32).at[: W0.shape[1], :].set(W0.T)
    z0, s0, q0 = _run_d1(g, nxrep, w0et)
    a0, c0 = _bn_coeffs(s0, q0, g0[None, :], b0[None, :], float(ROWS))

    z1, s1, q1 = _run_d2(z0, a0, c0, W1.T)
    a1, c1 = _bn_coeffs(s1, q1, g1[None, :], b1[None, :], float(ROWS))

    pooled = _run_d3(z1.reshape(B * S, NS, 64), a1[:, None, :], c1[:, None, :])

    new_xyz = nx3
    new_features = jnp.transpose(pooled.reshape(B, S, 64), (0, 2, 1))
    return new_xyz, new_features, inds.astype(jnp.int32)


# speedup vs baseline: 17.9498x; 1.2245x over previous
import functools

import jax
import jax.numpy as jnp
from jax import lax
from jax.experimental import pallas as pl
from jax.experimental.pallas import tpu as pltpu
from jax.experimental.pallas import tpu_sc as plsc

B = 4
N = 8192
S = 512
NS = 32
R2 = 0.2 * 0.2
CPAD = 32

NC = 2
NSUB = 16
NW = NC * NSUB
ROWS = B * S * NS
RPW = ROWS // NW
GCH = 128
NCHUNK = RPW // GCH


NR = N // 128
SR = S // 128


def _fps_body(xyz_ref, inds_ref, nx_ref):
    X = xyz_ref[0]
    Y = xyz_ref[1]
    Z = xyz_ref[2]
    niota = (lax.broadcasted_iota(jnp.int32, (B, NR, 128), 1) * 128
             + lax.broadcasted_iota(jnp.int32, (B, NR, 128), 2))
    siota = (lax.broadcasted_iota(jnp.int32, (B, SR, 128), 1) * 128
             + lax.broadcasted_iota(jnp.int32, (B, SR, 128), 2))

    XYZ = xyz_ref[...]

    def red2(x, fn):
        return fn(fn(x, axis=-2, keepdims=True), axis=-1, keepdims=True)

    def body(i, carry):
        dists, far, inds, nxx, nxy, nxz = carry
        m = niota == far
        cxyz = red2(jnp.where(m[None], XYZ, 0.0), jnp.sum)
        cx = cxyz[0]
        cy = cxyz[1]
        cz = cxyz[2]
        smi = (siota == i).astype(jnp.int32)
        smf = smi.astype(jnp.float32)
        inds = inds + smi * far
        nxx = nxx + smf * cx
        nxy = nxy + smf * cy
        nxz = nxz + smf * cz
        d = (X - cx) ** 2 + (Y - cy) ** 2 + (Z - cz) ** 2
        dists = jnp.minimum(dists, d)
        mx = red2(dists, jnp.max)
        c = (dists == mx).astype(jnp.int32)
        far = red2(N + c * (niota - N), jnp.min)
        return dists, far, inds, nxx, nxy, nxz

    zi = (niota[:, :SR] + siota) >> 31
    zf = zi.astype(jnp.float32)
    init = (
        (niota >> 31).astype(jnp.float32) + 1e10,
        niota[:, :1, :1] >> 31,
        zi,
        zf,
        zf,
        zf,
    )
    _, _, inds, nxx, nxy, nxz = lax.fori_loop(0, S, body, init)
    inds_ref[...] = inds
    nx_ref[0] = nxx
    nx_ref[1] = nxy
    nx_ref[2] = nxz


def _run_fps(xyz_t4):
    return pl.pallas_call(
        _fps_body,
        out_shape=[
            jax.ShapeDtypeStruct((B, SR, 128), jnp.int32),
            jax.ShapeDtypeStruct((3, B, SR, 128), jnp.float32),
        ],
    )(xyz_t4)


TS = 128


def _bq_body(xyz_ref, nx_ref, out_ref):
    b = pl.program_id(0)
    X = xyz_ref[0, 0:1, :]
    Y = xyz_ref[0, 1:2, :]
    Z = xyz_ref[0, 2:3, :]
    cx = nx_ref[0, :, 0:1]
    cy = nx_ref[0, :, 1:2]
    cz = nx_ref[0, :, 2:3]
    c2 = cx * cx + cy * cy + cz * cz
    x2 = X * X + Y * Y + Z * Z
    e = jnp.dot(nx_ref[0], xyz_ref[0],
                preferred_element_type=jnp.float32)
    d = (c2 + x2) - 2.0 * e
    niota = lax.broadcasted_iota(jnp.int32, (TS, N), 1).astype(jnp.float32)
    inr = (d <= R2).astype(jnp.float32)
    nf = jnp.float32(N)
    keyf = nf + inr * (niota - nf)
    goff = b * N

    def next_after(prevf):
        ph = prevf + 0.5
        t = ph - keyf
        tb = lax.bitcast_convert_type(t, jnp.int32)
        mb = jnp.min(tb, axis=1, keepdims=True)
        return ph - lax.bitcast_convert_type(mb, jnp.float32)

    cur = next_after(jnp.full((TS, 1), -1.0, jnp.float32))
    f0 = jnp.where(cur == nf, 0.0, cur)
    out_ref[0, :, 0:1] = f0.astype(jnp.int32) + goff
    prev = cur
    for k in range(1, NS):
        cand = next_after(prev)
        out_ref[0, :, k:k + 1] = (
            jnp.where(cand == nf, f0, cand).astype(jnp.int32) + goff)
        prev = cand


def _run_ball_query(xyz_bt, nx3):
    return pl.pallas_call(
        _bq_body,
        grid=(B, S // TS),
        in_specs=[
            pl.BlockSpec((1, 3, N), lambda b, st: (b, 0, 0)),
            pl.BlockSpec((1, TS, 3), lambda b, st: (b, st, 0)),
        ],
        out_specs=pl.BlockSpec((1, TS, NS), lambda b, st: (b, st, 0)),
        out_shape=jax.ShapeDtypeStruct((B, S, NS), jnp.int32),
    )(xyz_bt, nx3)


def _sc_gather_body(tf_hbm, idx_hbm, out_hbm, idx_v, rows_v, sem):
    wid = lax.axis_index("s") * NC + lax.axis_index("c")
    pltpu.sync_copy(idx_hbm.at[wid], idx_v)
    descs = []
    for c in range(NCHUNK):
        descs.append(
            pltpu.async_copy(
                tf_hbm.at[idx_v.at[c]],
                rows_v.at[pl.ds(c * GCH, GCH)],
                sem,
            )
        )
    for d in descs:
        d.wait()
    pltpu.sync_copy(rows_v, out_hbm.at[pl.ds(wid * RPW, RPW)])


def _run_sc_gather(tf, idx3):
    mesh = plsc.VectorSubcoreMesh(core_axis_name="c", subcore_axis_name="s")
    k = pl.kernel(
        _sc_gather_body,
        out_type=jax.ShapeDtypeStruct((ROWS, CPAD), jnp.float32),
        mesh=mesh,
        scratch_types=[
            pltpu.VMEM((NCHUNK, GCH), jnp.int32),
            pltpu.VMEM((RPW, CPAD), jnp.float32),
            pltpu.SemaphoreType.DMA,
        ],
        compiler_params=pltpu.CompilerParams(use_tc_tiling_on_sc=False),
    )
    return k(tf, idx3)


CH = 8192
NCH = ROWS // CH


def _d1_body(g_ref, nx_ref, w_ref, z_ref, s_ref, q_ref):
    i = pl.program_id(0)
    dg = g_ref[...] - nx_ref[...]
    z = jnp.dot(dg, w_ref[...], preferred_element_type=jnp.float32)
    z_ref[...] = z

    @pl.when(i == 0)
    def _():
        s_ref[...] = jnp.zeros_like(s_ref)
        q_ref[...] = jnp.zeros_like(q_ref)

    s_ref[...] += jnp.sum(z, axis=0, keepdims=True)
    q_ref[...] += jnp.sum(z * z, axis=0, keepdims=True)


def _run_d1(g, nxrep, w0et):
    return pl.pallas_call(
        _d1_body,
        grid=(NCH,),
        in_specs=[
            pl.BlockSpec((CH, CPAD), lambda i: (i, 0)),
            pl.BlockSpec((CH, CPAD), lambda i: (i, 0)),
            pl.BlockSpec((CPAD, 32), lambda i: (0, 0)),
        ],
        out_specs=[
            pl.BlockSpec((CH, 32), lambda i: (i, 0)),
            pl.BlockSpec((1, 32), lambda i: (0, 0)),
            pl.BlockSpec((1, 32), lambda i: (0, 0)),
        ],
        out_shape=[
            jax.ShapeDtypeStruct((ROWS, 32), jnp.float32),
            jax.ShapeDtypeStruct((1, 32), jnp.float32),
            jax.ShapeDtypeStruct((1, 32), jnp.float32),
        ],
    )(g, nxrep, w0et)


def _d2_body(z_ref, a_ref, c_ref, w_ref, z1_ref, s_ref, q_ref):
    i = pl.program_id(0)
    x = jnp.maximum(z_ref[...] * a_ref[...] + c_ref[...], 0.0)
    z1 = jnp.dot(x, w_ref[...], preferred_element_type=jnp.float32)
    z1_ref[...] = z1

    @pl.when(i == 0)
    def _():
        s_ref[...] = jnp.zeros_like(s_ref)
        q_ref[...] = jnp.zeros_like(q_ref)

    s_ref[...] += jnp.sum(z1, axis=0, keepdims=True)
    q_ref[...] += jnp.sum(z1 * z1, axis=0, keepdims=True)


def _run_d2(z0, a0, c0, w1t):
    return pl.pallas_call(
        _d2_body,
        grid=(NCH,),
        in_specs=[
            pl.BlockSpec((CH, 32), lambda i: (i, 0)),
            pl.BlockSpec((1, 32), lambda i: (0, 0)),
            pl.BlockSpec((1, 32), lambda i: (0, 0)),
            pl.BlockSpec((32, 64), lambda i: (0, 0)),
        ],
        out_specs=[
            pl.BlockSpec((CH, 64), lambda i: (i, 0)),
            pl.BlockSpec((1, 64), lambda i: (0, 0)),
            pl.BlockSpec((1, 64), lambda i: (0, 0)),
        ],
        out_shape=[
            jax.ShapeDtypeStruct((ROWS, 64), jnp.float32),
            jax.ShapeDtypeStruct((1, 64), jnp.float32),
            jax.ShapeDtypeStruct((1, 64), jnp.float32),
        ],
    )(z0, a0, c0, w1t)


def _d3_body(z_ref, a_ref, c_ref, o_ref):
    x = jnp.maximum(z_ref[...] * a_ref[...] + c_ref[...], 0.0)
    o_ref[...] = jnp.max(x, axis=1)


SCH = CH // NS


def _run_d3(z1g, a1, c1):
    return pl.pallas_call(
        _d3_body,
        grid=(NCH,),
        in_specs=[
            pl.BlockSpec((SCH, NS, 64), lambda i: (i, 0, 0)),
            pl.BlockSpec((1, 1, 64), lambda i: (0, 0, 0)),
            pl.BlockSpec((1, 1, 64), lambda i: (0, 0, 0)),
        ],
        out_specs=pl.BlockSpec((SCH, 64), lambda i: (i, 0)),
        out_shape=jax.ShapeDtypeStruct((B * S, 64), jnp.float32),
    )(z1g, a1, c1)


def _bn_coeffs(s, q, g, b, n):
    mean = s / n
    var = q / n - mean * mean
    a = g / jnp.sqrt(var + 1e-5)
    c = b - mean * a
    return a, c


@jax.jit
def kernel(xyz, features, W0, g0, b0, W1, g1, b1):
    xyz_t4 = jnp.transpose(xyz, (2, 0, 1)).reshape(3, B, NR, 128)
    inds4, nx4 = _run_fps(xyz_t4)
    inds = inds4.reshape(B, S)
    nx3 = jnp.transpose(nx4.reshape(3, B, S), (1, 2, 0))

    idx = _run_ball_query(jnp.transpose(xyz, (0, 2, 1)), nx3)
    idx3 = idx.reshape(NW, NCHUNK, GCH)

    table = jnp.concatenate(
        [xyz, features, jnp.zeros((B, N, CPAD - 3 - features.shape[-1]), jnp.float32)],
        axis=-1,
    ).reshape(B * N, CPAD)
    g = _run_sc_gather(table, idx3)

    nxrep = jnp.concatenate(
        [nx3, jnp.zeros((B, S, CPAD - 3), jnp.float32)], axis=-1
    )
    nxrep = jnp.broadcast_to(nxrep[:, :, None, :], (B, S, NS, CPAD)).reshape(
        ROWS, CPAD
    )

    w0et = jnp.zeros((CPAD, 32), jnp.float32).at[: W0.shape[1], :].set(W0.T)
    z0, s0, q0 = _run_d1(g, nxrep, w0et)
    a0, c0 = _bn_coeffs(s0, q0, g0[None, :], b0[None, :], float(ROWS))

    z1, s1, q1 = _run_d2(z0, a0, c0, W1.T)
    a1, c1 = _bn_coeffs(s1, q1, g1[None, :], b1[None, :], float(ROWS))

    pooled = _run_d3(z1.reshape(B * S, NS, 64), a1[:, None, :], c1[:, None, :])

    new_xyz = nx3
    new_features = jnp.transpose(pooled.reshape(B, S, 64), (0, 2, 1))
    return new_xyz, new_features, inds.astype(jnp.int32)
